# baseline (speedup 1.0000x reference)
import functools

import jax
import jax.numpy as jnp
from jax import lax
from jax.experimental import pallas as pl
from jax.experimental.pallas import tpu as pltpu
from jax.experimental.pallas import tpu_sc as plsc

N = 10000
E = 320000
IN_DIM = 128
HID = 256
EMB = 64

NC = 2
NS = 16
C = 128
CH = 160
EPS = CH * C
EPAD = NS * EPS
NP = 10240
HR = NP // C
RPS = NP // NS
CZ = 128
NWB = RPS // CZ
KB = 40
NB = CH // KB

_f32 = jnp.float32


def _make_sc_l1():
  mesh = plsc.VectorSubcoreMesh(core_axis_name="c", subcore_axis_name="s")

  out_type = [jax.ShapeDtypeStruct((NP, IN_DIM), _f32),
              jax.ShapeDtypeStruct((NP, IN_DIM), _f32)]

  scratch = [
      pltpu.VMEM((KB, C), jnp.int32),
      pltpu.VMEM((KB, C), jnp.int32),
      pltpu.VMEM((C, IN_DIM), _f32),
      pltpu.VMEM((C, IN_DIM), _f32),
      pltpu.SemaphoreType.DMA,
      pltpu.SemaphoreType.DMA,
      pltpu.VMEM_SHARED((NP, IN_DIM), _f32),
  ]

  @functools.partial(pl.kernel, out_type=out_type, mesh=mesh,
                     scratch_types=scratch,
                     compiler_params=pltpu.CompilerParams(
                         use_tc_tiling_on_sc=False))
  def sc_kernel(xu, xi, su, du, si, di,
                agg_u, agg_i,
                srcv, dstv, rowb0, rowb1, gsem0, gsem1,
                acc):
    rowb = (rowb0, rowb1)
    gsem = (gsem0, gsem1)
    c = lax.axis_index("c")
    s = lax.axis_index("s")
    base = s * RPS

    zeros16 = jnp.zeros((16,), _f32)

    def fill_rowb0(i, carry):
      rowb0[i // 8, pl.ds((i % 8) * 16, 16)] = zeros16
      return carry

    lax.fori_loop(0, C * IN_DIM // 16, fill_rowb0, 0)

    for k in range(NWB):
      pltpu.sync_copy(rowb0, acc.at[pl.ds(base + k * CZ, CZ)])

    plsc.subcore_barrier()

    def run(x_hbm, src_hbm, dst_hbm):
      def block(kb, carry):
        pltpu.sync_copy(src_hbm.at[s, pl.ds(kb * KB, KB)], srcv)
        pltpu.sync_copy(dst_hbm.at[s, pl.ds(kb * KB, KB)], dstv)
        pltpu.async_copy(x_hbm.at[srcv.at[0]], rowb[0], gsem[0])

        def step(g, carry2):
          for b in range(2):
            j = 2 * g + b
            nxt = (b + 1) % 2

            @pl.when(j + 1 < KB)
            def _():
              pltpu.async_copy(x_hbm.at[srcv.at[j + 1]], rowb[nxt],
                               gsem[nxt])

            pltpu.make_async_copy(x_hbm.at[srcv.at[j]], rowb[b],
                                  gsem[b]).wait()
            pltpu.sync_copy(rowb[b], acc.at[dstv.at[j]], add=True)
          return carry2

        lax.fori_loop(0, KB // 2, step, 0)
        return carry

      lax.fori_loop(0, NB, block, 0)

    @pl.when(c == 0)
    def _():
      run(xu, su, du)

    @pl.when(c == 1)
    def _():
      run(xi, si, di)

    plsc.subcore_barrier()

    def writeback(agg_hbm):
      for k in range(NWB):
        off = base + k * CZ
        pltpu.sync_copy(acc.at[pl.ds(off, CZ)], rowb0)
        pltpu.sync_copy(rowb0, agg_hbm.at[pl.ds(off, CZ)])

    @pl.when(c == 0)
    def _():
      writeback(agg_u)

    @pl.when(c == 1)
    def _():
      writeback(agg_i)

  return sc_kernel


def _make_sc_deg():
  mesh = plsc.VectorSubcoreMesh(core_axis_name="c", subcore_axis_name="s")

  out_type = [jax.ShapeDtypeStruct((NS, HR, C), _f32),
              jax.ShapeDtypeStruct((NS, HR, C), _f32),
              jax.ShapeDtypeStruct((HR, C), _f32),
              jax.ShapeDtypeStruct((HR, C), _f32)]

  scratch = [
      pltpu.VMEM((CH, C), jnp.int32),
      pltpu.VMEM((HR, C), _f32),
      pltpu.VMEM((8, C), _f32),
      pltpu.VMEM((8, C), _f32),
  ]

  @functools.partial(pl.kernel, out_type=out_type, mesh=mesh,
                     scratch_types=scratch,
                     compiler_params=pltpu.CompilerParams(
                         needs_layout_passes=False))
  def sc_kernel(du, di, dp_u, dp_i, deg_u, deg_i, dstv, hist, racc, rbuf):
    c = lax.axis_index("c")
    s = lax.axis_index("s")

    zeros16 = jnp.zeros((16,), _f32)
    ones16 = jnp.ones((16,), _f32)

    def fill_hist(i, carry):
      hist[i // 8, pl.ds((i % 8) * 16, 16)] = zeros16
      return carry

    lax.fori_loop(0, HR * C // 16, fill_hist, 0)

    @pl.when(c == 0)
    def _():
      pltpu.sync_copy(du.at[s], dstv)

    @pl.when(c == 1)
    def _():
      pltpu.sync_copy(di.at[s], dstv)

    def count(j, carry):
      for u in range(C // 16):
        idx = dstv[j, pl.ds(u * 16, 16)]
        plsc.addupdate_scatter(
            hist, [lax.shift_right_logical(idx, 7),
                   lax.bitwise_and(idx, 127)], ones16)
      return carry

    lax.fori_loop(0, CH, count, 0)

    def publish(dp_hbm):
      pltpu.sync_copy(hist, dp_hbm.at[s])

    @pl.when(c == 0)
    def _():
      publish(dp_u)

    @pl.when(c == 1)
    def _():
      publish(dp_i)

    plsc.subcore_barrier()

    @pl.when(s < NS - 6)
    def _():
      def fill_racc(i, carry):
        racc[i // 8, pl.ds((i % 8) * 16, 16)] = zeros16
        return carry

      lax.fori_loop(0, 8 * C // 16, fill_racc, 0)

      def reduce_from(dp_hbm):
        for t in range(NS):
          pltpu.sync_copy(dp_hbm.at[t, pl.ds(8 * s, 8)], rbuf)

          def add_rows(i, carry):
            sl = pl.ds((i % 8) * 16, 16)
            racc[i // 8, sl] = racc[i // 8, sl] + rbuf[i // 8, sl]
            return carry

          lax.fori_loop(0, 8 * C // 16, add_rows, 0)

      @pl.when(c == 0)
      def _():
        reduce_from(dp_u)
        pltpu.sync_copy(racc, deg_u.at[pl.ds(8 * s, 8)])

      @pl.when(c == 1)
      def _():
        reduce_from(dp_i)
        pltpu.sync_copy(racc, deg_i.at[pl.ds(8 * s, 8)])

  return sc_kernel


_sc_segsum = _make_sc_l1()
_sc_deg = _make_sc_deg()


_BT = 1024
_GT = NP // _BT
_HB = _BT // C


def _dot_t(a, w):
  return lax.dot_general(a, w, (((1,), (1,)), ((), ())),
                         preferred_element_type=_f32)


def _tc_mid_body(agg_u, deg_u, xu, agg_i, deg_i, xi,
                 uWl1, ubl1, uWr1, uWl2, ubl2, uWr2,
                 iWl1, ibl1, iWr1, iWl2, ibl2, iWr2,
                 pcat, r_u, r_i):
  def enc(agg, deg, x, Wl1, bl1, Wr1, Wl2, bl2, Wr2, r_out):
    d = jnp.maximum(deg[...], 1.0)
    mean = agg[...] / d
    h = jnp.maximum(_dot_t(mean, Wl1[...]) + bl1[...] + _dot_t(x[...], Wr1[...]),
                    0.0)
    r_out[...] = _dot_t(h, Wr2[...]) + bl2[...]
    return _dot_t(h, Wl2[...])

  p_u = enc(agg_u, deg_u, xu, uWl1, ubl1, uWr1, uWl2, ubl2, uWr2, r_u)
  p_i = enc(agg_i, deg_i, xi, iWl1, ibl1, iWr1, iWl2, ibl2, iWr2, r_i)
  pcat[...] = jnp.concatenate([p_u, p_i], axis=1)


def _row_spec(d):
  return pl.BlockSpec((_BT, d), lambda i: (i, 0))


def _full_spec(shape):
  nd = len(shape)
  return pl.BlockSpec(shape, lambda i: (0,) * nd)


def _tc_mid(agg_u, deg_u, xu, agg_i, deg_i, xi, wu, wi):
  w_specs = [_full_spec(w.shape) for w in (wu + wi)]
  return pl.pallas_call(
      _tc_mid_body,
      grid=(_GT,),
      in_specs=[_row_spec(IN_DIM), _row_spec(1), _row_spec(IN_DIM),
                _row_spec(IN_DIM), _row_spec(1), _row_spec(IN_DIM)] + w_specs,
      out_specs=[_row_spec(IN_DIM), _row_spec(EMB), _row_spec(EMB)],
      out_shape=[jax.ShapeDtypeStruct((NP, IN_DIM), _f32),
                 jax.ShapeDtypeStruct((NP, EMB), _f32),
                 jax.ShapeDtypeStruct((NP, EMB), _f32)],
  )(agg_u, deg_u, xu, agg_i, deg_i, xi, *wu, *wi)


def _tc_head_body(a2u, deg_u, ru, a2i, deg_i, ri, sW, sb, out):
  eu = a2u[...][:, :EMB] / jnp.maximum(deg_u[...], 1.0) + ru[...]
  ei = a2i[...][:, EMB:] / jnp.maximum(deg_i[...], 1.0) + ri[...]
  w = sW[...]
  z = _dot_t(eu, w[:, :EMB]) + _dot_t(ei, w[:, EMB:]) + sb[...]
  out[...] = 1.0 / (1.0 + jnp.exp(-z))


def _tc_head(a2u, deg_u, ru, a2i, deg_i, ri, sW, sb):
  return pl.pallas_call(
      _tc_head_body,
      grid=(_GT,),
      in_specs=[_row_spec(IN_DIM), _row_spec(1), _row_spec(EMB),
                _row_spec(IN_DIM), _row_spec(1), _row_spec(EMB),
                _full_spec((1, 2 * EMB)), _full_spec((1, 1))],
      out_specs=_row_spec(1),
      out_shape=jax.ShapeDtypeStruct((NP, 1), _f32),
  )(a2u, deg_u, ru, a2i, deg_i, ri, sW, sb)


def kernel(user_x, item_x, user_edge_index, item_edge_index,
           u_Wl1, u_bl1, u_Wr1, u_Wl2, u_bl2, u_Wr2,
           i_Wl1, i_bl1, i_Wr1, i_Wl2, i_bl2, i_Wr2,
           s_W, s_b):
  npad = EPAD - E

  def edges(ei):
    src = jnp.concatenate(
        [ei[0].astype(jnp.int32), jnp.zeros((npad,), jnp.int32)])
    dst = jnp.concatenate(
        [ei[1].astype(jnp.int32), jnp.full((npad,), NP - 1, jnp.int32)])
    return src.reshape(NS, CH, C), dst.reshape(NS, CH, C)

  su, du = edges(user_edge_index)
  si, di = edges(item_edge_index)

  xu = jnp.pad(user_x, ((0, NP - N), (0, 0)))
  xi = jnp.pad(item_x, ((0, NP - N), (0, 0)))

  _, _, deg80_u, deg80_i = _sc_deg(du, di)
  deg_u = deg80_u.reshape(NP, 1)
  deg_i = deg80_i.reshape(NP, 1)

  agg_u, agg_i = _sc_segsum(xu, xi, su, du, si, di)

  wu = (u_Wl1, u_bl1.reshape(1, HID), u_Wr1,
        u_Wl2, u_bl2.reshape(1, EMB), u_Wr2)
  wi = (i_Wl1, i_bl1.reshape(1, HID), i_Wr1,
        i_Wl2, i_bl2.reshape(1, EMB), i_Wr2)
  pcat, r_u, r_i = _tc_mid(agg_u, deg_u, xu, agg_i, deg_i, xi, wu, wi)

  agg2_u, agg2_i = _sc_segsum(pcat, pcat, su, du, si, di)

  out = _tc_head(agg2_u, deg_u, r_u, agg2_i, deg_i, r_i,
                 s_W, s_b.reshape(1, 1))
  return out[:N]


# speedup vs baseline: 2.6255x; 2.6255x over previous
import functools

import jax
import jax.numpy as jnp
from jax import lax
from jax.experimental import pallas as pl
from jax.experimental.pallas import tpu as pltpu
from jax.experimental.pallas import tpu_sc as plsc

N = 10000
E = 320000
IN_DIM = 128
HID = 256
EMB = 64

NC = 2
NS = 16
C = 125
EPS = E // NS
CH = EPS // C
NP = 10240
RPS = NP // NS
CZ = 128
NWB = RPS // CZ

_f32 = jnp.float32


D = EMB


def _make_sc_segsum(num_passes, with_deg):
  mesh = plsc.VectorSubcoreMesh(core_axis_name="c", subcore_axis_name="s")

  out_type = [jax.ShapeDtypeStruct((NP, D), _f32)] * (2 * num_passes)
  if with_deg:
    out_type += [jax.ShapeDtypeStruct((NP, 16), _f32),
                 jax.ShapeDtypeStruct((NP, 16), _f32)]

  scratch = [
      pltpu.VMEM((CH, C), jnp.int32),
      pltpu.VMEM((CH, C), jnp.int32),
      pltpu.VMEM((C, D), _f32),
      pltpu.VMEM((C, D), _f32),
      pltpu.VMEM((CZ, D), _f32),
      pltpu.SemaphoreType.DMA,
      pltpu.SemaphoreType.DMA,
      pltpu.VMEM_SHARED((NP, D), _f32),
  ]
  if with_deg:
    scratch += [
        pltpu.VMEM((C, 16), _f32),
        pltpu.VMEM((CZ, 16), _f32),
        pltpu.VMEM_SHARED((NP, 16), _f32),
    ]

  @functools.partial(pl.kernel, out_type=out_type, mesh=mesh,
                     scratch_types=scratch,
                     compiler_params=pltpu.CompilerParams(
                         use_tc_tiling_on_sc=False))
  def sc_kernel(*refs):
    pos = 0
    xs_u = refs[pos:pos + num_passes]; pos += num_passes
    xs_i = refs[pos:pos + num_passes]; pos += num_passes
    su, du, si, di, zfeat_hbm = refs[pos:pos + 5]; pos += 5
    if with_deg:
      z16_hbm, ones_hbm = refs[pos:pos + 2]; pos += 2
    aggs_u = refs[pos:pos + num_passes]; pos += num_passes
    aggs_i = refs[pos:pos + num_passes]; pos += num_passes
    if with_deg:
      deg_u, deg_i = refs[pos:pos + 2]; pos += 2
    srcv, dstv, rowb0, rowb1, zfv, gsem0, gsem1, acc = refs[pos:pos + 8]
    pos += 8
    if with_deg:
      onesv, z16v, dacc = refs[pos:pos + 3]; pos += 3
    rowb = (rowb0, rowb1)
    gsem = (gsem0, gsem1)

    c = lax.axis_index("c")
    s = lax.axis_index("s")
    base = s * RPS

    pltpu.sync_copy(zfeat_hbm, zfv)
    if with_deg:
      pltpu.sync_copy(z16_hbm, z16v)
      pltpu.sync_copy(ones_hbm, onesv)

    @pl.when(c == 0)
    def _():
      pltpu.sync_copy(su.at[s], srcv)
      pltpu.sync_copy(du.at[s], dstv)

    @pl.when(c == 1)
    def _():
      pltpu.sync_copy(si.at[s], srcv)
      pltpu.sync_copy(di.at[s], dstv)

    for p in range(num_passes):
      first = (p == 0)
      for k in range(NWB):
        pltpu.sync_copy(zfv, acc.at[pl.ds(base + k * CZ, CZ)])
        if with_deg and first:
          pltpu.sync_copy(z16v, dacc.at[pl.ds(base + k * CZ, CZ)])
      plsc.subcore_barrier()

      def run(x_hbm, do_deg):
        pltpu.async_copy(x_hbm.at[srcv.at[0]], rowb[0], gsem[0])

        def step(g, carry):
          for b in range(2):
            j = 2 * g + b
            nxt = (b + 1) % 2

            @pl.when(j + 1 < CH)
            def _():
              pltpu.async_copy(x_hbm.at[srcv.at[j + 1]], rowb[nxt], gsem[nxt])

            pltpu.make_async_copy(x_hbm.at[srcv.at[j]], rowb[b],
                                  gsem[b]).wait()
            pltpu.sync_copy(rowb[b], acc.at[dstv.at[j]], add=True)
            if do_deg:
              pltpu.sync_copy(onesv, dacc.at[dstv.at[j]], add=True)
          return carry

        lax.fori_loop(0, CH // 2, step, 0)

      @pl.when(c == 0)
      def _():
        run(xs_u[p], with_deg and first)

      @pl.when(c == 1)
      def _():
        run(xs_i[p], with_deg and first)

      plsc.subcore_barrier()

      def writeback(agg_hbm, deg_hbm):
        for k in range(NWB):
          off = base + k * CZ
          pltpu.sync_copy(acc.at[pl.ds(off, CZ)], zfv)
          pltpu.sync_copy(zfv, agg_hbm.at[pl.ds(off, CZ)])
          if deg_hbm is not None:
            pltpu.sync_copy(dacc.at[pl.ds(off, CZ)], z16v)
            pltpu.sync_copy(z16v, deg_hbm.at[pl.ds(off, CZ)])

      @pl.when(c == 0)
      def _():
        writeback(aggs_u[p], deg_u if (with_deg and first) else None)

      @pl.when(c == 1)
      def _():
        writeback(aggs_i[p], deg_i if (with_deg and first) else None)

      if p + 1 < num_passes:
        pltpu.sync_copy(zfeat_hbm, zfv)

  return sc_kernel


_sc_segsum_l1 = _make_sc_segsum(2, with_deg=False)
_sc_segsum_l2 = _make_sc_segsum(1, with_deg=False)


CD = 128
CHD = 160
EPADD = NS * CHD * CD
HR = NP // CD


def _make_sc_deg():
  mesh = plsc.VectorSubcoreMesh(core_axis_name="c", subcore_axis_name="s")

  out_type = [jax.ShapeDtypeStruct((NS, HR, CD), _f32),
              jax.ShapeDtypeStruct((NS, HR, CD), _f32),
              jax.ShapeDtypeStruct((HR, CD), _f32),
              jax.ShapeDtypeStruct((HR, CD), _f32)]

  scratch = [
      pltpu.VMEM((CHD, CD), jnp.int32),
      pltpu.VMEM((HR, CD), _f32),
      pltpu.VMEM((8, CD), _f32),
      pltpu.VMEM((8, CD), _f32),
  ]

  @functools.partial(pl.kernel, out_type=out_type, mesh=mesh,
                     scratch_types=scratch,
                     compiler_params=pltpu.CompilerParams(
                         needs_layout_passes=False,
                         use_tc_tiling_on_sc=False))
  def sc_kernel(du, di, dp_u, dp_i, deg_u, deg_i, dstv, hist, racc, rbuf):
    c = lax.axis_index("c")
    s = lax.axis_index("s")

    zeros16 = jnp.zeros((16,), _f32)
    ones16 = jnp.ones((16,), _f32)

    def fill_hist(i, carry):
      hist[i // 8, pl.ds((i % 8) * 16, 16)] = zeros16
      return carry

    lax.fori_loop(0, HR * CD // 16, fill_hist, 0)

    @pl.when(c == 0)
    def _():
      pltpu.sync_copy(du.at[s], dstv)

    @pl.when(c == 1)
    def _():
      pltpu.sync_copy(di.at[s], dstv)

    def count(j, carry):
      for u in range(CD // 16):
        idx = dstv[j, pl.ds(u * 16, 16)]
        plsc.addupdate_scatter(
            hist, [lax.shift_right_logical(idx, 7),
                   lax.bitwise_and(idx, 127)], ones16)
      return carry

    lax.fori_loop(0, CHD, count, 0)

    @pl.when(c == 0)
    def _():
      pltpu.sync_copy(hist, dp_u.at[s])

    @pl.when(c == 1)
    def _():
      pltpu.sync_copy(hist, dp_i.at[s])

    plsc.subcore_barrier()

    @pl.when(s < HR // 8)
    def _():
      def fill_racc(i, carry):
        racc[i // 8, pl.ds((i % 8) * 16, 16)] = zeros16
        return carry

      lax.fori_loop(0, 8 * CD // 16, fill_racc, 0)

      def reduce_from(dp_hbm):
        for t in range(NS):
          pltpu.sync_copy(dp_hbm.at[t, pl.ds(8 * s, 8)], rbuf)

          def add_rows(i, carry):
            sl = pl.ds((i % 8) * 16, 16)
            racc[i // 8, sl] = racc[i // 8, sl] + rbuf[i // 8, sl]
            return carry

          lax.fori_loop(0, 8 * CD // 16, add_rows, 0)

      @pl.when(c == 0)
      def _():
        reduce_from(dp_u)
        pltpu.sync_copy(racc, deg_u.at[pl.ds(8 * s, 8)])

      @pl.when(c == 1)
      def _():
        reduce_from(dp_i)
        pltpu.sync_copy(racc, deg_i.at[pl.ds(8 * s, 8)])

  return sc_kernel


_sc_deg = _make_sc_deg()


_BT = 1000
_GT = N // _BT


def _dot_t(a, w):
  return lax.dot_general(a, w, (((1,), (1,)), ((), ())),
                         preferred_element_type=_f32)


def _tc_mid_body(agg_u0, agg_u1, deg_u, xu, agg_i0, agg_i1, deg_i, xi,
                 uWl1, ubl1, uWr1, uWl2, ubl2, uWr2,
                 iWl1, ibl1, iWr1, iWl2, ibl2, iWr2,
                 p_u, r_u, p_i, r_i):
  def enc(agg0, agg1, deg, x, Wl1, bl1, Wr1, Wl2, bl2, Wr2, p_out, r_out):
    d = jnp.maximum(deg[...], 1.0)
    mean = jnp.concatenate([agg0[...], agg1[...]], axis=1) / d
    h = jnp.maximum(_dot_t(mean, Wl1[...]) + bl1[...] + _dot_t(x[...], Wr1[...]),
                    0.0)
    p_out[...] = _dot_t(h, Wl2[...])
    r_out[...] = _dot_t(h, Wr2[...]) + bl2[...]

  enc(agg_u0, agg_u1, deg_u, xu, uWl1, ubl1, uWr1, uWl2, ubl2, uWr2, p_u, r_u)
  enc(agg_i0, agg_i1, deg_i, xi, iWl1, ibl1, iWr1, iWl2, ibl2, iWr2, p_i, r_i)


def _row_spec(d):
  return pl.BlockSpec((_BT, d), lambda i: (i, 0))


def _full_spec(shape):
  nd = len(shape)
  return pl.BlockSpec(shape, lambda i: (0,) * nd)


def _tc_mid(agg_u0, agg_u1, deg_u, xu, agg_i0, agg_i1, deg_i, xi, wu, wi):
  w_specs = [_full_spec(w.shape) for w in (wu + wi)]
  return pl.pallas_call(
      _tc_mid_body,
      grid=(_GT,),
      in_specs=[_row_spec(EMB), _row_spec(EMB), _row_spec(1), _row_spec(IN_DIM),
                _row_spec(EMB), _row_spec(EMB), _row_spec(1), _row_spec(IN_DIM)]
               + w_specs,
      out_specs=[_row_spec(EMB)] * 4,
      out_shape=[jax.ShapeDtypeStruct((N, EMB), _f32)] * 4,
  )(agg_u0, agg_u1, deg_u, xu, agg_i0, agg_i1, deg_i, xi, *wu, *wi)


def _tc_head_body(a2u, deg_u, ru, a2i, deg_i, ri, sW, sb, out):
  eu = a2u[...] / jnp.maximum(deg_u[...], 1.0) + ru[...]
  ei = a2i[...] / jnp.maximum(deg_i[...], 1.0) + ri[...]
  w = sW[...]
  z = _dot_t(eu, w[:, :EMB]) + _dot_t(ei, w[:, EMB:]) + sb[...]
  out[...] = 1.0 / (1.0 + jnp.exp(-z))


def _tc_head(a2u, deg_u, ru, a2i, deg_i, ri, sW, sb):
  return pl.pallas_call(
      _tc_head_body,
      grid=(_GT,),
      in_specs=[_row_spec(EMB), _row_spec(1), _row_spec(EMB),
                _row_spec(EMB), _row_spec(1), _row_spec(EMB),
                _full_spec((1, 2 * EMB)), _full_spec((1, 1))],
      out_specs=_row_spec(1),
      out_shape=jax.ShapeDtypeStruct((N, 1), _f32),
  )(a2u, deg_u, ru, a2i, deg_i, ri, sW, sb)


def kernel(user_x, item_x, user_edge_index, item_edge_index,
           u_Wl1, u_bl1, u_Wr1, u_Wl2, u_bl2, u_Wr2,
           i_Wl1, i_bl1, i_Wr1, i_Wl2, i_bl2, i_Wr2,
           s_W, s_b):
  def edges(ei):
    src = ei[0].astype(jnp.int32).reshape(NS, CH, C)
    dst = ei[1].astype(jnp.int32).reshape(NS, CH, C)
    return src, dst

  su, du = edges(user_edge_index)
  si, di = edges(item_edge_index)

  def deg_edges(ei):
    dst = jnp.concatenate([ei[1].astype(jnp.int32),
                           jnp.full((EPADD - E,), NP - 1, jnp.int32)])
    return dst.reshape(NS, CHD, CD)

  dud = deg_edges(user_edge_index)
  did = deg_edges(item_edge_index)
  _, _, deg80_u, deg80_i = _sc_deg(dud, did)
  deg_u = deg80_u.reshape(NP, 1)
  deg_i = deg80_i.reshape(NP, 1)

  z64 = jnp.zeros((CZ, D), _f32)

  xu0, xu1 = user_x[:, :D], user_x[:, D:]
  xi0, xi1 = item_x[:, :D], item_x[:, D:]

  agg_u0, agg_u1, agg_i0, agg_i1 = _sc_segsum_l1(
      xu0, xu1, xi0, xi1, su, du, si, di, z64)

  wu = (u_Wl1, u_bl1.reshape(1, HID), u_Wr1,
        u_Wl2, u_bl2.reshape(1, EMB), u_Wr2)
  wi = (i_Wl1, i_bl1.reshape(1, HID), i_Wr1,
        i_Wl2, i_bl2.reshape(1, EMB), i_Wr2)
  p_u, r_u, p_i, r_i = _tc_mid(agg_u0, agg_u1, deg_u, user_x,
                               agg_i0, agg_i1, deg_i, item_x, wu, wi)

  agg2_u, agg2_i = _sc_segsum_l2(p_u, p_i, su, du, si, di, z64)

  return _tc_head(agg2_u, deg_u, r_u, agg2_i, deg_i, r_i,
                  s_W, s_b.reshape(1, 1))


# speedup vs baseline: 2.7374x; 1.0426x over previous
import functools

import jax
import jax.numpy as jnp
from jax import lax
from jax.experimental import pallas as pl
from jax.experimental.pallas import tpu as pltpu
from jax.experimental.pallas import tpu_sc as plsc

N = 10000
E = 320000
IN_DIM = 128
HID = 256
EMB = 64

NC = 2
NS = 16
NW = NC * NS
C = 125
EPS = E // NW
CH = EPS // C
NP = 10240
RPS = NP // NS
CZ = 128
NWB = RPS // CZ

_f32 = jnp.float32
D = EMB


def _make_sc_segsum(num_passes, with_deg):
  mesh = plsc.VectorSubcoreMesh(core_axis_name="c", subcore_axis_name="s")

  out_type = [jax.ShapeDtypeStruct((NC, NP, D), _f32)] * num_passes
  if with_deg:
    out_type += [jax.ShapeDtypeStruct((NC, NP, 16), _f32)]

  scratch = [
      pltpu.VMEM((CH, C), jnp.int32),
      pltpu.VMEM((CH, C), jnp.int32),
      pltpu.VMEM((C, D), _f32),
      pltpu.VMEM((C, D), _f32),
      pltpu.VMEM((CZ, D), _f32),
      pltpu.SemaphoreType.DMA,
      pltpu.SemaphoreType.DMA,
      pltpu.VMEM_SHARED((NP, D), _f32),
  ]
  if with_deg:
    scratch += [
        pltpu.VMEM((C, 16), _f32),
        pltpu.VMEM((CZ, 16), _f32),
        pltpu.VMEM_SHARED((NP, 16), _f32),
    ]

  @functools.partial(pl.kernel, out_type=out_type, mesh=mesh,
                     scratch_types=scratch,
                     compiler_params=pltpu.CompilerParams(
                         use_tc_tiling_on_sc=False))
  def sc_kernel(*refs):
    pos = 0
    xs = refs[pos:pos + num_passes]; pos += num_passes
    src4, dst4, zfeat_hbm = refs[pos:pos + 3]; pos += 3
    if with_deg:
      z16_hbm, ones_hbm = refs[pos:pos + 2]; pos += 2
    parts = refs[pos:pos + num_passes]; pos += num_passes
    if with_deg:
      deg_out = refs[pos]; pos += 1
    srcv, dstv, rowb0, rowb1, zfv, gsem0, gsem1, acc = refs[pos:pos + 8]
    pos += 8
    if with_deg:
      onesv, z16v, dacc = refs[pos:pos + 3]; pos += 3
    rowb = (rowb0, rowb1)
    gsem = (gsem0, gsem1)

    c = lax.axis_index("c")
    s = lax.axis_index("s")
    base = s * RPS

    pltpu.sync_copy(zfeat_hbm, zfv)
    if with_deg:
      pltpu.sync_copy(z16_hbm, z16v)
      pltpu.sync_copy(ones_hbm, onesv)

    pltpu.sync_copy(src4.at[c, s], srcv)
    pltpu.sync_copy(dst4.at[c, s], dstv)

    for p in range(num_passes):
      first = (p == 0)
      for k in range(NWB):
        pltpu.sync_copy(zfv, acc.at[pl.ds(base + k * CZ, CZ)])
        if with_deg and first:
          pltpu.sync_copy(z16v, dacc.at[pl.ds(base + k * CZ, CZ)])
      plsc.subcore_barrier()

      x_hbm = xs[p]
      pltpu.async_copy(x_hbm.at[srcv.at[0]], rowb[0], gsem[0])

      def step(g, carry):
        for b in range(2):
          j = 2 * g + b
          nxt = (b + 1) % 2

          @pl.when(j + 1 < CH)
          def _():
            pltpu.async_copy(x_hbm.at[srcv.at[j + 1]], rowb[nxt], gsem[nxt])

          pltpu.make_async_copy(x_hbm.at[srcv.at[j]], rowb[b],
                                gsem[b]).wait()
          pltpu.sync_copy(rowb[b], acc.at[dstv.at[j]], add=True)
          if with_deg and first:
            pltpu.sync_copy(onesv, dacc.at[dstv.at[j]], add=True)
        return carry

      lax.fori_loop(0, CH // 2, step, 0)

      plsc.subcore_barrier()

      for k in range(NWB):
        off = base + k * CZ
        pltpu.sync_copy(acc.at[pl.ds(off, CZ)], zfv)
        pltpu.sync_copy(zfv, parts[p].at[c, pl.ds(off, CZ)])
        if with_deg and first:
          pltpu.sync_copy(dacc.at[pl.ds(off, CZ)], z16v)
          pltpu.sync_copy(z16v, deg_out.at[c, pl.ds(off, CZ)])

      if p + 1 < num_passes:
        pltpu.sync_copy(zfeat_hbm, zfv)

  return sc_kernel


_sc_segsum_l1 = _make_sc_segsum(2, with_deg=True)
_sc_segsum_l2 = _make_sc_segsum(1, with_deg=False)


_BT = 1000
_GT = N // _BT


def _dot_t(a, w):
  return lax.dot_general(a, w, (((1,), (1,)), ((), ())),
                         preferred_element_type=_f32)


def _tc_mid_body(a0, a1, deg, x, Wl1, bl1, Wr1, Wl2, bl2, Wr2,
                 p_out, r_out, d_out):
  degv = deg[...]
  d = jnp.maximum(degv[0][:, :1] + degv[1][:, :1], 1.0)
  d_out[...] = d
  a0v = a0[...]
  a1v = a1[...]
  mean = jnp.concatenate([a0v[0] + a0v[1], a1v[0] + a1v[1]], axis=1) / d
  h = jnp.maximum(_dot_t(mean, Wl1[...]) + bl1[...] + _dot_t(x[...], Wr1[...]),
                  0.0)
  p_out[...] = _dot_t(h, Wl2[...])
  r_out[...] = _dot_t(h, Wr2[...]) + bl2[...]


def _row_spec(d):
  return pl.BlockSpec((_BT, d), lambda i: (i, 0))


def _part_spec(d):
  return pl.BlockSpec((NC, _BT, d), lambda i: (0, i, 0))


def _full_spec(shape):
  nd = len(shape)
  return pl.BlockSpec(shape, lambda i: (0,) * nd)


def _tc_mid(a0, a1, deg, x, w):
  w_specs = [_full_spec(wi.shape) for wi in w]
  return pl.pallas_call(
      _tc_mid_body,
      grid=(_GT,),
      in_specs=[_part_spec(D), _part_spec(D), _part_spec(16),
                _row_spec(IN_DIM)] + w_specs,
      out_specs=[_row_spec(EMB), _row_spec(EMB), _row_spec(1)],
      out_shape=[jax.ShapeDtypeStruct((N, EMB), _f32),
                 jax.ShapeDtypeStruct((N, EMB), _f32),
                 jax.ShapeDtypeStruct((N, 1), _f32)],
  )(a0, a1, deg, x, *w)


def _tc_head_body(a2u, du, ru, a2i, di, ri, sW, sb, out):
  a2uv = a2u[...]
  a2iv = a2i[...]
  eu = (a2uv[0] + a2uv[1]) / du[...] + ru[...]
  ei = (a2iv[0] + a2iv[1]) / di[...] + ri[...]
  w = sW[...]
  z = _dot_t(eu, w[:, :EMB]) + _dot_t(ei, w[:, EMB:]) + sb[...]
  out[...] = 1.0 / (1.0 + jnp.exp(-z))


def _tc_head(a2u, du, ru, a2i, di, ri, sW, sb):
  return pl.pallas_call(
      _tc_head_body,
      grid=(_GT,),
      in_specs=[_part_spec(D), _row_spec(1), _row_spec(EMB),
                _part_spec(D), _row_spec(1), _row_spec(EMB),
                _full_spec((1, 2 * EMB)), _full_spec((1, 1))],
      out_specs=_row_spec(1),
      out_shape=jax.ShapeDtypeStruct((N, 1), _f32),
  )(a2u, du, ru, a2i, di, ri, sW, sb)


def kernel(user_x, item_x, user_edge_index, item_edge_index,
           u_Wl1, u_bl1, u_Wr1, u_Wl2, u_bl2, u_Wr2,
           i_Wl1, i_bl1, i_Wr1, i_Wl2, i_bl2, i_Wr2,
           s_W, s_b):
  def edges(ei):
    src = ei[0].astype(jnp.int32).reshape(NC, NS, CH, C)
    dst = ei[1].astype(jnp.int32).reshape(NC, NS, CH, C)
    return src, dst

  su, du = edges(user_edge_index)
  si, di = edges(item_edge_index)

  z64 = jnp.zeros((CZ, D), _f32)
  z16 = jnp.zeros((CZ, 16), _f32)
  ones16 = jnp.ones((C, 16), _f32)

  xu0, xu1 = user_x[:, :D], user_x[:, D:]
  xi0, xi1 = item_x[:, :D], item_x[:, D:]

  wu = (u_Wl1, u_bl1.reshape(1, HID), u_Wr1,
        u_Wl2, u_bl2.reshape(1, EMB), u_Wr2)
  wi = (i_Wl1, i_bl1.reshape(1, HID), i_Wr1,
        i_Wl2, i_bl2.reshape(1, EMB), i_Wr2)

  a_u0, a_u1, deg_u = _sc_segsum_l1(xu0, xu1, su, du, z64, z16, ones16)
  p_u, r_u, d_u = _tc_mid(a_u0, a_u1, deg_u, user_x, wu)

  a_i0, a_i1, deg_i = _sc_segsum_l1(xi0, xi1, si, di, z64, z16, ones16)
  p_i, r_i, d_i = _tc_mid(a_i0, a_i1, deg_i, item_x, wi)

  a2_u, = _sc_segsum_l2(p_u, su, du, z64)
  a2_i, = _sc_segsum_l2(p_i, si, di, z64)

  return _tc_head(a2_u, d_u, r_u, a2_i, d_i, r_i, s_W, s_b.reshape(1, 1))


# speedup vs baseline: 2.7419x; 1.0017x over previous
import functools

import jax
import jax.numpy as jnp
from jax import lax
from jax.experimental import pallas as pl
from jax.experimental.pallas import tpu as pltpu
from jax.experimental.pallas import tpu_sc as plsc

N = 10000
E = 320000
IN_DIM = 128
HID = 256
EMB = 64

NC = 2
NS = 16
NW = NC * NS
C = 125
EPS = E // NW
CH = EPS // C
NP = 10240
RPS = NP // NS
CZ = 128
NWB = RPS // CZ

_f32 = jnp.float32
D = EMB


def _make_sc_segsum(num_passes, with_deg):
  mesh = plsc.VectorSubcoreMesh(core_axis_name="c", subcore_axis_name="s")

  out_type = [jax.ShapeDtypeStruct((NC, NP, D), _f32)] * num_passes
  if with_deg:
    out_type += [jax.ShapeDtypeStruct((NC, NP, 16), _f32)]

  scratch = [
      pltpu.VMEM((CH, C), jnp.int32),
      pltpu.VMEM((CH, C), jnp.int32),
      pltpu.VMEM((C, D), _f32),
      pltpu.VMEM((C, D), _f32),
      pltpu.VMEM((CZ, D), _f32),
      pltpu.SemaphoreType.DMA,
      pltpu.SemaphoreType.DMA,
      pltpu.VMEM_SHARED((NP, D), _f32),
  ]
  if with_deg:
    scratch += [
        pltpu.VMEM((C, 16), _f32),
        pltpu.VMEM((CZ, 16), _f32),
        pltpu.VMEM_SHARED((NP, 16), _f32),
    ]

  @functools.partial(pl.kernel, out_type=out_type, mesh=mesh,
                     scratch_types=scratch,
                     compiler_params=pltpu.CompilerParams(
                         use_tc_tiling_on_sc=False))
  def sc_kernel(*refs):
    pos = 0
    xs = refs[pos:pos + num_passes]; pos += num_passes
    src4, dst4, zfeat_hbm = refs[pos:pos + 3]; pos += 3
    if with_deg:
      z16_hbm, ones_hbm = refs[pos:pos + 2]; pos += 2
    parts = refs[pos:pos + num_passes]; pos += num_passes
    if with_deg:
      deg_out = refs[pos]; pos += 1
    srcv, dstv, rowb0, rowb1, zfv, gsem0, gsem1, acc = refs[pos:pos + 8]
    pos += 8
    if with_deg:
      onesv, z16v, dacc = refs[pos:pos + 3]; pos += 3
    rowb = (rowb0, rowb1)
    gsem = (gsem0, gsem1)

    c = lax.axis_index("c")
    s = lax.axis_index("s")
    base = s * RPS

    pltpu.sync_copy(zfeat_hbm, zfv)
    if with_deg:
      pltpu.sync_copy(z16_hbm, z16v)
      pltpu.sync_copy(ones_hbm, onesv)

    pltpu.sync_copy(src4.at[c, s], srcv)
    pltpu.sync_copy(dst4.at[c, s], dstv)

    for p in range(num_passes):
      first = (p == 0)
      for k in range(NWB):
        pltpu.sync_copy(zfv, acc.at[pl.ds(base + k * CZ, CZ)])
        if with_deg and first:
          pltpu.sync_copy(z16v, dacc.at[pl.ds(base + k * CZ, CZ)])
      plsc.subcore_barrier()

      x_hbm = xs[p]
      pltpu.async_copy(x_hbm.at[srcv.at[0]], rowb[0], gsem[0])

      def step(g, carry):
        for b in range(2):
          j = 2 * g + b
          nxt = (b + 1) % 2

          @pl.when(j + 1 < CH)
          def _():
            pltpu.async_copy(x_hbm.at[srcv.at[j + 1]], rowb[nxt], gsem[nxt])

          pltpu.make_async_copy(x_hbm.at[srcv.at[j]], rowb[b],
                                gsem[b]).wait()
          pltpu.sync_copy(rowb[b], acc.at[dstv.at[j]], add=True)
          if with_deg and first:
            pltpu.sync_copy(onesv, dacc.at[dstv.at[j]], add=True)
        return carry

      lax.fori_loop(0, CH // 2, step, 0)

      plsc.subcore_barrier()

      for k in range(NWB):
        off = base + k * CZ
        pltpu.sync_copy(acc.at[pl.ds(off, CZ)], zfv)
        pltpu.sync_copy(zfv, parts[p].at[c, pl.ds(off, CZ)])
        if with_deg and first:
          pltpu.sync_copy(dacc.at[pl.ds(off, CZ)], z16v)
          pltpu.sync_copy(z16v, deg_out.at[c, pl.ds(off, CZ)])

      if p + 1 < num_passes:
        pltpu.sync_copy(zfeat_hbm, zfv)

  return sc_kernel


_sc_segsum_l1 = _make_sc_segsum(2, with_deg=True)
_sc_segsum_l2 = _make_sc_segsum(1, with_deg=False)


_BT = 1000
_GT = N // _BT


def _dot_t(a, w):
  return lax.dot_general(a, w, (((1,), (1,)), ((), ())),
                         preferred_element_type=_f32)


def _tc_mid_body(a0, a1, deg, x, Wl1, bl1, Wr1, Wl2, bl2, Wr2,
                 p_out, r_out, d_out):
  degv = deg[...]
  d = jnp.maximum(degv[0][:, :1] + degv[1][:, :1], 1.0)
  d_out[...] = d
  a0v = a0[...]
  a1v = a1[...]
  mean = jnp.concatenate([a0v[0] + a0v[1], a1v[0] + a1v[1]], axis=1) / d
  h = jnp.maximum(_dot_t(mean, Wl1[...]) + bl1[...] + _dot_t(x[...], Wr1[...]),
                  0.0)
  p_out[...] = _dot_t(h, Wl2[...])
  r_out[...] = _dot_t(h, Wr2[...]) + bl2[...]


def _row_spec(d):
  return pl.BlockSpec((_BT, d), lambda i: (i, 0))


def _part_spec(d):
  return pl.BlockSpec((NC, _BT, d), lambda i: (0, i, 0))


def _full_spec(shape):
  nd = len(shape)
  return pl.BlockSpec(shape, lambda i: (0,) * nd)


def _tc_mid(a0, a1, deg, x, w):
  w_specs = [_full_spec(wi.shape) for wi in w]
  return pl.pallas_call(
      _tc_mid_body,
      grid=(_GT,),
      in_specs=[_part_spec(D), _part_spec(D), _part_spec(16),
                _row_spec(IN_DIM)] + w_specs,
      out_specs=[_row_spec(EMB), _row_spec(EMB), _row_spec(1)],
      out_shape=[jax.ShapeDtypeStruct((N, EMB), _f32),
                 jax.ShapeDtypeStruct((N, EMB), _f32),
                 jax.ShapeDtypeStruct((N, 1), _f32)],
  )(a0, a1, deg, x, *w)


def _tc_head_body(a2u, du, ru, a2i, di, ri, sW, sb, out):
  a2uv = a2u[...]
  a2iv = a2i[...]
  eu = (a2uv[0] + a2uv[1]) / du[...] + ru[...]
  ei = (a2iv[0] + a2iv[1]) / di[...] + ri[...]
  w = sW[...]
  z = _dot_t(eu, w[:, :EMB]) + _dot_t(ei, w[:, EMB:]) + sb[...]
  out[...] = 1.0 / (1.0 + jnp.exp(-z))


def _tc_head(a2u, du, ru, a2i, di, ri, sW, sb):
  return pl.pallas_call(
      _tc_head_body,
      grid=(_GT,),
      in_specs=[_part_spec(D), _row_spec(1), _row_spec(EMB),
                _part_spec(D), _row_spec(1), _row_spec(EMB),
                _full_spec((1, 2 * EMB)), _full_spec((1, 1))],
      out_specs=_row_spec(1),
      out_shape=jax.ShapeDtypeStruct((N, 1), _f32),
  )(a2u, du, ru, a2i, di, ri, sW, sb)


def kernel(user_x, item_x, user_edge_index, item_edge_index,
           u_Wl1, u_bl1, u_Wr1, u_Wl2, u_bl2, u_Wr2,
           i_Wl1, i_bl1, i_Wr1, i_Wl2, i_bl2, i_Wr2,
           s_W, s_b):
  def edges(ei):
    src = ei[0].astype(jnp.int32).reshape(NC, NS, CH, C)
    dst = ei[1].astype(jnp.int32).reshape(NC, NS, CH, C)
    return src, dst

  su, du = edges(user_edge_index)
  si, di = edges(item_edge_index)

  z64 = jnp.zeros((CZ, D), _f32)
  z16 = jnp.zeros((CZ, 16), _f32)
  ones16 = jnp.ones((C, 16), _f32)

  xu0, xu1 = user_x[:, :D], user_x[:, D:]
  xi0, xi1 = item_x[:, :D], item_x[:, D:]

  wu = (u_Wl1, u_bl1.reshape(1, HID), u_Wr1,
        u_Wl2, u_bl2.reshape(1, EMB), u_Wr2)
  wi = (i_Wl1, i_bl1.reshape(1, HID), i_Wr1,
        i_Wl2, i_bl2.reshape(1, EMB), i_Wr2)

  a_u0, a_u1, deg_u = _sc_segsum_l1(xu0, xu1, su, du, z64, z16, ones16)
  a_i0, a_i1, deg_i = _sc_segsum_l1(xi0, xi1, si, di, z64, z16, ones16)
  p_u, r_u, d_u = _tc_mid(a_u0, a_u1, deg_u, user_x, wu)
  p_i, r_i, d_i = _tc_mid(a_i0, a_i1, deg_i, item_x, wi)

  a2_u, = _sc_segsum_l2(p_u, su, du, z64)
  a2_i, = _sc_segsum_l2(p_i, si, di, z64)

  return _tc_head(a2_u, d_u, r_u, a2_i, d_i, r_i, s_W, s_b.reshape(1, 1))
